# baseline (speedup 1.0000x reference)
import functools

import jax
import jax.numpy as jnp
from jax import lax
from jax.experimental import pallas as pl
from jax.experimental.pallas import tpu as pltpu
from jax.experimental.pallas import tpu_sc as plsc

N = 10000
NP = 10240
D = 128
E = 320000
NCLS = 40

NC = 2
NS = 16
NW = NC * NS
CH = 128
NCHUNK = 80
EP = NW * NCHUNK * CH
NT = NP // NS

def _sage_body(x_hbm, src_hbm, dst_hbm, zrow_hbm, zdeg_hbm, ones_hbm,
               aggp_hbm, degp_hbm,
               src_v, dst_v, rows_v, ones_v, agg_s, deg_s):
    c = lax.axis_index("c")
    s = lax.axis_index("s")
    w = s * NC + c
    base = s * NT
    pltpu.sync_copy(src_hbm.at[w], src_v)
    pltpu.sync_copy(dst_hbm.at[w], dst_v)
    pltpu.sync_copy(ones_hbm, ones_v)
    pltpu.sync_copy(zrow_hbm, agg_s.at[pl.ds(base, NT)])
    pltpu.sync_copy(zdeg_hbm, deg_s.at[pl.ds(base, NT)])
    plsc.subcore_barrier()

    def body(j, carry):
        pltpu.sync_copy(x_hbm.at[src_v.at[j]], rows_v)
        pltpu.sync_copy(rows_v, agg_s.at[dst_v.at[j]], add=True)
        pltpu.sync_copy(ones_v, deg_s.at[dst_v.at[j]], add=True)
        return carry

    lax.fori_loop(0, NCHUNK, body, 0)
    plsc.subcore_barrier()
    pltpu.sync_copy(agg_s.at[pl.ds(base, NT)], aggp_hbm.at[c, pl.ds(base, NT)])
    pltpu.sync_copy(deg_s.at[pl.ds(base, NT)], degp_hbm.at[c, pl.ds(base, NT)])


@functools.cache
def _get_sage_agg():
    mesh = plsc.VectorSubcoreMesh(core_axis_name="c", subcore_axis_name="s",
                                  num_cores=NC, num_subcores=NS)
    return pl.kernel(
        _sage_body,
        out_type=(jax.ShapeDtypeStruct((NC, NP, D), jnp.float32),
                  jax.ShapeDtypeStruct((NC, NP), jnp.float32)),
        mesh=mesh,
        scratch_types=[
            pltpu.VMEM((NCHUNK, CH), jnp.int32),
            pltpu.VMEM((NCHUNK, CH), jnp.int32),
            pltpu.VMEM((CH, D), jnp.float32),
            pltpu.VMEM((CH,), jnp.float32),
            pltpu.VMEM_SHARED((NP, D), jnp.float32),
            pltpu.VMEM_SHARED((NP,), jnp.float32),
        ],
    )


BR = 2048
NB = NP // BR


def _mask_rows(i, val):
    rows = i * BR + lax.broadcasted_iota(jnp.int32, val.shape, 0)
    return jnp.where(rows < N, val, 0.0)


def _mean_agg(agg0, agg1, deg0, deg1):
    deg = jnp.maximum(deg0[...] + deg1[...], 1.0)
    return (agg0[...] + agg1[...]) / deg


def _dot(a, b):
    return jnp.dot(a, b, preferred_element_type=jnp.float32)


def _tc1_body(xh, posf, Wp, bp, We0, be0, We1, be1, Wi, bi, x0_o, hyp_o):
    i = pl.program_id(0)
    p = _dot(posf[...], Wp[...]) + bp[...]
    h = jax.nn.relu(_dot(p, We0[...]) + be0[...])
    h = jax.nn.relu(_dot(h, We1[...]) + be1[...])
    hyp_o[...] = jnp.tanh(h)
    x0_o[...] = _mask_rows(i, _dot(xh[...], Wi[...]) + bi[...])


def _tc2_body(agg0, agg1, deg0, deg1, x0, hyp, Wl, bl, Wr, x1_o):
    i = pl.program_id(0)
    agg = _mean_agg(agg0, agg1, deg0, deg1)
    x = _dot(agg, Wl[...]) + bl[...] + _dot(x0[...], Wr[...]) + hyp[...]
    x1_o[...] = _mask_rows(i, jax.nn.relu(x))


def _tc3_body(agg0, agg1, deg0, deg1, x1, Wl, bl, Wr, Wlast, blast,
              emb_o, logp_o):
    agg = _mean_agg(agg0, agg1, deg0, deg1)
    x2 = _dot(agg, Wl[...]) + bl[...] + _dot(x1[...], Wr[...])
    e = _dot(x2, Wlast[...]) + blast[...]
    m = jnp.max(e, axis=1, keepdims=True)
    lse = m + jnp.log(jnp.sum(jnp.exp(e - m), axis=1, keepdims=True))
    emb_o[...] = e
    logp_o[...] = e - lse


def _rowspec(cols):
    return pl.BlockSpec((BR, cols), lambda i: (i, 0))


def _fullspec(r, c):
    return pl.BlockSpec((r, c), lambda i: (0, 0))


_tc1 = pl.pallas_call(
    _tc1_body,
    grid=(NB,),
    in_specs=[
        _rowspec(D), _rowspec(16),
        _fullspec(16, D), _fullspec(1, D),
        _fullspec(D, D), _fullspec(1, D),
        _fullspec(D, D), _fullspec(1, D),
        _fullspec(D, D), _fullspec(1, D),
    ],
    out_specs=[_rowspec(D), _rowspec(D)],
    out_shape=[jax.ShapeDtypeStruct((NP, D), jnp.float32),
               jax.ShapeDtypeStruct((NP, D), jnp.float32)],
)

_tc2 = pl.pallas_call(
    _tc2_body,
    grid=(NB,),
    in_specs=[
        _rowspec(D), _rowspec(D), _rowspec(1), _rowspec(1),
        _rowspec(D), _rowspec(D),
        _fullspec(D, D), _fullspec(1, D), _fullspec(D, D),
    ],
    out_specs=[_rowspec(D)],
    out_shape=[jax.ShapeDtypeStruct((NP, D), jnp.float32)],
)

_tc3 = pl.pallas_call(
    _tc3_body,
    grid=(NB,),
    in_specs=[
        _rowspec(D), _rowspec(D), _rowspec(1), _rowspec(1),
        _rowspec(D),
        _fullspec(D, D), _fullspec(1, D), _fullspec(D, D),
        _fullspec(D, NCLS), _fullspec(1, NCLS),
    ],
    out_specs=[_rowspec(NCLS), _rowspec(NCLS)],
    out_shape=[jax.ShapeDtypeStruct((NP, NCLS), jnp.float32),
               jax.ShapeDtypeStruct((NP, NCLS), jnp.float32)],
)


@jax.jit
def kernel(x_h, adj, edge_index, pos_feat, Wp, bp, We0, be0, We1, be1,
           Wi, bi, Wl0, bl0, Wr0, Wl1, bl1, Wr1, Wlast, blast):
    del adj
    src = edge_index[0]
    dst = edge_index[1]
    pad = EP - E
    srcp = jnp.concatenate([src, jnp.full((pad,), N, jnp.int32)]).reshape(NW, NCHUNK, CH)
    dstp = jnp.concatenate([dst, jnp.full((pad,), N, jnp.int32)]).reshape(NW, NCHUNK, CH)
    xh_p = jnp.pad(x_h, ((0, NP - N), (0, 0)))
    pos_p = jnp.pad(pos_feat, ((0, NP - N), (0, 0)))
    zrow = jnp.zeros((NT, D), jnp.float32)
    zdeg = jnp.zeros((NT,), jnp.float32)
    ones = jnp.ones((CH,), jnp.float32)
    r = lambda v: v.reshape(1, -1)

    sage_agg = _get_sage_agg()
    x0, hyp = _tc1(xh_p, pos_p, Wp, r(bp), We0, r(be0), We1, r(be1), Wi, r(bi))
    aggp, degp = sage_agg(x0, srcp, dstp, zrow, zdeg, ones)
    deg0 = degp[0].reshape(NP, 1)
    deg1 = degp[1].reshape(NP, 1)
    x1, = _tc2(aggp[0], aggp[1], deg0, deg1, x0, hyp, Wl0, r(bl0), Wr0)
    aggp2, _ = sage_agg(x1, srcp, dstp, zrow, zdeg, ones)
    emb, logp = _tc3(aggp2[0], aggp2[1], deg0, deg1, x1,
                     Wl1, r(bl1), Wr1, Wlast, r(blast))
    return emb[:N], logp[:N]


# speedup vs baseline: 1.5199x; 1.5199x over previous
import functools

import jax
import jax.numpy as jnp
from jax import lax
from jax.experimental import pallas as pl
from jax.experimental.pallas import tpu as pltpu
from jax.experimental.pallas import tpu_sc as plsc

N = 10000
NP = 10240
D = 128
E = 320000
NCLS = 40

NC = 2
NS = 16
NW = NC * NS
CH = 128
NCHUNK = -(-E // (NW * CH))
EP = NW * NCHUNK * CH
NT = NP // NS

def _sage_body(x_hbm, src_hbm, dst_hbm, zrow_hbm, zdeg_hbm, ones_hbm,
               aggp_hbm, degp_hbm,
               src_v, dst_v, rows_v, ones_v, agg_s, deg_s):
    c = lax.axis_index("c")
    s = lax.axis_index("s")
    w = s * NC + c
    base = s * NT
    pltpu.sync_copy(src_hbm.at[w], src_v)
    pltpu.sync_copy(dst_hbm.at[w], dst_v)
    pltpu.sync_copy(ones_hbm, ones_v)
    pltpu.sync_copy(zrow_hbm, agg_s.at[pl.ds(base, NT)])
    pltpu.sync_copy(zdeg_hbm, deg_s.at[pl.ds(base, NT)])
    plsc.subcore_barrier()

    def body(j, carry):
        pltpu.sync_copy(x_hbm.at[src_v.at[j]], rows_v)
        pltpu.sync_copy(rows_v, agg_s.at[dst_v.at[j]], add=True)
        pltpu.sync_copy(ones_v, deg_s.at[dst_v.at[j]], add=True)
        return carry

    lax.fori_loop(0, NCHUNK, body, 0)
    plsc.subcore_barrier()
    pltpu.sync_copy(agg_s.at[pl.ds(base, NT)], aggp_hbm.at[c, pl.ds(base, NT)])
    pltpu.sync_copy(deg_s.at[pl.ds(base, NT)], degp_hbm.at[c, pl.ds(base, NT)])


@functools.cache
def _get_sage_agg():
    mesh = plsc.VectorSubcoreMesh(core_axis_name="c", subcore_axis_name="s",
                                  num_cores=NC, num_subcores=NS)
    return pl.kernel(
        _sage_body,
        out_type=(jax.ShapeDtypeStruct((NC, NP, D), jnp.float32),
                  jax.ShapeDtypeStruct((NC, NP), jnp.float32)),
        mesh=mesh,
        scratch_types=[
            pltpu.VMEM((NCHUNK, CH), jnp.int32),
            pltpu.VMEM((NCHUNK, CH), jnp.int32),
            pltpu.VMEM((CH, D), jnp.float32),
            pltpu.VMEM((CH,), jnp.float32),
            pltpu.VMEM_SHARED((NP, D), jnp.float32),
            pltpu.VMEM_SHARED((NP,), jnp.float32),
        ],
    )


BR = 2048
NB = NP // BR


def _mask_rows(i, val):
    rows = i * BR + lax.broadcasted_iota(jnp.int32, val.shape, 0)
    return jnp.where(rows < N, val, 0.0)


def _mean_agg(agg0, agg1, deg0, deg1):
    deg = jnp.maximum(deg0[...] + deg1[...], 1.0)
    return (agg0[...] + agg1[...]) / deg


def _dot(a, b):
    return jnp.dot(a, b, preferred_element_type=jnp.float32)


def _tc1_body(xh, posf, Wp, bp, We0, be0, We1, be1, Wi, bi, x0_o, hyp_o):
    i = pl.program_id(0)
    p = _dot(posf[...], Wp[...]) + bp[...]
    h = jax.nn.relu(_dot(p, We0[...]) + be0[...])
    h = jax.nn.relu(_dot(h, We1[...]) + be1[...])
    hyp_o[...] = jnp.tanh(h)
    x0_o[...] = _mask_rows(i, _dot(xh[...], Wi[...]) + bi[...])


def _tc2_body(agg0, agg1, deg0, deg1, x0, hyp, Wl, bl, Wr, x1_o):
    i = pl.program_id(0)
    agg = _mean_agg(agg0, agg1, deg0, deg1)
    x = _dot(agg, Wl[...]) + bl[...] + _dot(x0[...], Wr[...]) + hyp[...]
    x1_o[...] = _mask_rows(i, jax.nn.relu(x))


def _tc3_body(agg0, agg1, deg0, deg1, x1, Wl, bl, Wr, Wlast, blast,
              emb_o, logp_o):
    agg = _mean_agg(agg0, agg1, deg0, deg1)
    x2 = _dot(agg, Wl[...]) + bl[...] + _dot(x1[...], Wr[...])
    e = _dot(x2, Wlast[...]) + blast[...]
    m = jnp.max(e, axis=1, keepdims=True)
    lse = m + jnp.log(jnp.sum(jnp.exp(e - m), axis=1, keepdims=True))
    emb_o[...] = e
    logp_o[...] = e - lse


def _rowspec(cols):
    return pl.BlockSpec((BR, cols), lambda i: (i, 0))


def _fullspec(r, c):
    return pl.BlockSpec((r, c), lambda i: (0, 0))


_tc1 = pl.pallas_call(
    _tc1_body,
    grid=(NB,),
    in_specs=[
        _rowspec(D), _rowspec(16),
        _fullspec(16, D), _fullspec(1, D),
        _fullspec(D, D), _fullspec(1, D),
        _fullspec(D, D), _fullspec(1, D),
        _fullspec(D, D), _fullspec(1, D),
    ],
    out_specs=[_rowspec(D), _rowspec(D)],
    out_shape=[jax.ShapeDtypeStruct((NP, D), jnp.float32),
               jax.ShapeDtypeStruct((NP, D), jnp.float32)],
)

_tc2 = pl.pallas_call(
    _tc2_body,
    grid=(NB,),
    in_specs=[
        _rowspec(D), _rowspec(D), _rowspec(1), _rowspec(1),
        _rowspec(D), _rowspec(D),
        _fullspec(D, D), _fullspec(1, D), _fullspec(D, D),
    ],
    out_specs=[_rowspec(D)],
    out_shape=[jax.ShapeDtypeStruct((NP, D), jnp.float32)],
)

_tc3 = pl.pallas_call(
    _tc3_body,
    grid=(NB,),
    in_specs=[
        _rowspec(D), _rowspec(D), _rowspec(1), _rowspec(1),
        _rowspec(D),
        _fullspec(D, D), _fullspec(1, D), _fullspec(D, D),
        _fullspec(D, NCLS), _fullspec(1, NCLS),
    ],
    out_specs=[_rowspec(NCLS), _rowspec(NCLS)],
    out_shape=[jax.ShapeDtypeStruct((NP, NCLS), jnp.float32),
               jax.ShapeDtypeStruct((NP, NCLS), jnp.float32)],
)


@jax.jit
def kernel(x_h, adj, edge_index, pos_feat, Wp, bp, We0, be0, We1, be1,
           Wi, bi, Wl0, bl0, Wr0, Wl1, bl1, Wr1, Wlast, blast):
    del adj
    src = edge_index[0]
    dst = edge_index[1]
    pad = EP - E
    srcp = jnp.concatenate([src, jnp.full((pad,), N, jnp.int32)]).reshape(NW, NCHUNK, CH)
    dstp = jnp.concatenate([dst, jnp.full((pad,), N, jnp.int32)]).reshape(NW, NCHUNK, CH)
    xh_p = jnp.pad(x_h, ((0, NP - N), (0, 0)))
    pos_p = jnp.pad(pos_feat, ((0, NP - N), (0, 0)))
    zrow = jnp.zeros((NT, D), jnp.float32)
    zdeg = jnp.zeros((NT,), jnp.float32)
    ones = jnp.ones((CH,), jnp.float32)
    r = lambda v: v.reshape(1, -1)

    sage_agg = _get_sage_agg()
    x0, hyp = _tc1(xh_p, pos_p, Wp, r(bp), We0, r(be0), We1, r(be1), Wi, r(bi))
    aggp, degp = sage_agg(x0, srcp, dstp, zrow, zdeg, ones)
    deg0 = degp[0].reshape(NP, 1)
    deg1 = degp[1].reshape(NP, 1)
    x1, = _tc2(aggp[0], aggp[1], deg0, deg1, x0, hyp, Wl0, r(bl0), Wr0)
    aggp2, _ = sage_agg(x1, srcp, dstp, zrow, zdeg, ones)
    emb, logp = _tc3(aggp2[0], aggp2[1], deg0, deg1, x1,
                     Wl1, r(bl1), Wr1, Wlast, r(blast))
    return emb[:N], logp[:N]
